# Initial kernel scaffold; baseline (speedup 1.0000x reference)
import functools

import jax
import jax.numpy as jnp
from jax import lax
from jax.experimental import pallas as pl
from jax.experimental.pallas import tpu as pltpu
from jax.experimental.pallas import tpu_sc as plsc

N = 10000
E = 320000
D = 128
H = 128
FC = 256
C = 2
G = 64

NC = 2
NS = 16
NW = NC * NS
L = 16
EPW = E // NW
K = 400
NCH = EPW // K
NP = 10240
RPT = NP // NS

_mesh = plsc.VectorSubcoreMesh(core_axis_name="c", subcore_axis_name="s")


@functools.partial(
    pl.kernel,
    out_type=[
        jax.ShapeDtypeStruct((NC, NP, H), jnp.float32),
        jax.ShapeDtypeStruct((NC, NP), jnp.float32),
    ],
    mesh=_mesh,
    scratch_types=[
        pltpu.VMEM((K,), jnp.int32),
        pltpu.VMEM((K,), jnp.int32),
        pltpu.VMEM((K, H), jnp.float32),
        pltpu.VMEM((K, H), jnp.float32),
        pltpu.VMEM((K,), jnp.float32),
        pltpu.VMEM((H,), jnp.float32),
        pltpu.VMEM((640,), jnp.float32),
        pltpu.VMEM_SHARED((NP, H), jnp.float32),
        pltpu.VMEM_SHARED((NP,), jnp.float32),
        pltpu.SemaphoreType.DMA,
        pltpu.SemaphoreType.DMA,
    ],
)
def _edge_pass(xl_hbm, xr_hbm, src_hbm, dst_hbm, att_hbm,
               acc_out, den_out,
               sidx, didx, A, B, W, attv, Z, accs, dens, sem1, sem2):
    c = lax.axis_index("c")
    s = lax.axis_index("s")
    wid = s * NC + c
    iota = lax.broadcasted_iota(jnp.int32, (L,), 0)
    zeros16 = jnp.zeros((L,), jnp.float32)

    pltpu.sync_copy(att_hbm, attv)

    def _zero_a(r, _):
        for j in range(H // L):
            A[r, pl.ds(j * L, L)] = zeros16
        return 0
    lax.fori_loop(0, K, _zero_a, 0)

    def _zero_z(i, _):
        Z[pl.ds(i * L, L)] = zeros16
        return 0
    lax.fori_loop(0, RPT // L, _zero_z, 0)

    rbase = s * RPT
    pltpu.sync_copy(A, accs.at[pl.ds(rbase, K)])
    pltpu.sync_copy(A.at[pl.ds(0, RPT - K)], accs.at[pl.ds(rbase + K, RPT - K)])
    pltpu.sync_copy(Z, dens.at[pl.ds(rbase, RPT)])
    plsc.subcore_barrier()

    def _chunk(g, _):
        ebase = wid * EPW + g * K
        pltpu.sync_copy(src_hbm.at[pl.ds(ebase, K)], sidx)
        pltpu.sync_copy(dst_hbm.at[pl.ds(ebase, K)], didx)
        cp1 = pltpu.async_copy(xl_hbm.at[sidx], A, sem1)
        cp2 = pltpu.async_copy(xr_hbm.at[didx], B, sem2)
        cp1.wait()
        cp2.wait()

        def _group(eg, _):
            rows = eg * L + iota

            def _dot(d, alpha):
                cols = jnp.full((L,), d, jnp.int32)
                av = plsc.load_gather(A, [rows, cols])
                bv = plsc.load_gather(B, [rows, cols])
                t = av + bv
                t = jnp.maximum(t, t * 0.2)
                return alpha + t * attv[d]

            alpha = lax.fori_loop(0, H, _dot, zeros16)
            wv = jnp.exp(alpha)
            W[pl.ds(eg * L, L)] = wv

            def _scale(d, _):
                cols = jnp.full((L,), d, jnp.int32)
                av = plsc.load_gather(A, [rows, cols])
                plsc.store_scatter(A, [rows, cols], av * wv)
                return 0

            lax.fori_loop(0, H, _scale, 0)
            return 0

        lax.fori_loop(0, K // L, _group, 0)

        pltpu.sync_copy(A, accs.at[didx], add=True)
        pltpu.sync_copy(W, dens.at[didx], add=True)
        return 0

    lax.fori_loop(0, NCH, _chunk, 0)
    plsc.subcore_barrier()

    pltpu.sync_copy(accs.at[pl.ds(rbase, RPT)], acc_out.at[c, pl.ds(rbase, RPT)])
    pltpu.sync_copy(dens.at[pl.ds(rbase, RPT)], den_out.at[c, pl.ds(rbase, RPT)])


def _tc0_body(x_ref, wl_ref, wr_ref, xl_ref, xr_ref):
    xb = x_ref[...]
    xl_ref[...] = jnp.dot(xb, wl_ref[...], preferred_element_type=jnp.float32)
    xr_ref[...] = jnp.dot(xb, wr_ref[...], preferred_element_type=jnp.float32)


def _tc0(x, Wl, Wr):
    return pl.pallas_call(
        _tc0_body,
        out_shape=[jax.ShapeDtypeStruct((N, H), jnp.float32),
                   jax.ShapeDtypeStruct((N, H), jnp.float32)],
    )(x, Wl, Wr)


def _combine(acc_ref, den_ref, b_ref):
    acc = acc_ref[0, :N, :] + acc_ref[1, :N, :]
    den = den_ref[0, :N] + den_ref[1, :N]
    v = acc / (den + 1e-16)[:, None] + b_ref[...][None, :]
    return jnp.where(v > 0.0, v, jnp.expm1(v))


def _tcmid_body(acc_ref, den_ref, b_ref, wl_ref, wr_ref, xl_ref, xr_ref):
    h = _combine(acc_ref, den_ref, b_ref)
    xl_ref[...] = jnp.dot(h, wl_ref[...], preferred_element_type=jnp.float32)
    xr_ref[...] = jnp.dot(h, wr_ref[...], preferred_element_type=jnp.float32)


def _tcmid(acc, den, b, Wl, Wr):
    return pl.pallas_call(
        _tcmid_body,
        out_shape=[jax.ShapeDtypeStruct((N, H), jnp.float32),
                   jax.ShapeDtypeStruct((N, H), jnp.float32)],
    )(acc, den, b, Wl, Wr)


def _tcfinal_body(acc_ref, den_ref, b_ref, batch_ref, fc1w_ref, fc1b_ref,
                  fc2w_ref, fc2b_ref, out_ref):
    h = _combine(acc_ref, den_ref, b_ref)
    batch = batch_ref[:N]
    onehot = (batch[:, None] == lax.broadcasted_iota(jnp.int32, (N, G), 1))
    onehot = onehot.astype(jnp.float32)
    sums = jnp.dot(onehot.T, h, preferred_element_type=jnp.float32)
    counts = jnp.sum(onehot, axis=0)
    pooled = sums / jnp.maximum(counts, 1.0)[:, None]
    z = jnp.dot(pooled, fc1w_ref[...], preferred_element_type=jnp.float32)
    z = jnp.maximum(z + fc1b_ref[...][None, :], 0.0)
    z = jnp.dot(z, fc2w_ref[...], preferred_element_type=jnp.float32)
    z = z + fc2b_ref[...][None, :]
    m = jnp.max(z, axis=1, keepdims=True)
    lse = m + jnp.log(jnp.sum(jnp.exp(z - m), axis=1, keepdims=True))
    out_ref[...] = z - lse


def _tcfinal(acc, den, b, batch_pad, fc1_W, fc1_b, fc2_W, fc2_b):
    return pl.pallas_call(
        _tcfinal_body,
        out_shape=jax.ShapeDtypeStruct((G, C), jnp.float32),
    )(acc, den, b, batch_pad, fc1_W, fc1_b, fc2_W, fc2_b)


def kernel(x, edge_index, batch, Wl0, Wr0, att0, b0, Wl1, Wr1, att1, b1,
           Wl2, Wr2, att2, b2, fc1_W, fc1_b, fc2_W, fc2_b):
    src = edge_index[0]
    dst = edge_index[1]
    batch_pad = jnp.pad(batch, (0, NP - N), constant_values=G)

    xl, xr = _tc0(x, Wl0, Wr0)
    acc, den = _edge_pass(xl, xr, src, dst, att0)
    xl, xr = _tcmid(acc, den, b0, Wl1, Wr1)
    acc, den = _edge_pass(xl, xr, src, dst, att1)
    xl, xr = _tcmid(acc, den, b1, Wl2, Wr2)
    acc, den = _edge_pass(xl, xr, src, dst, att2)
    return _tcfinal(acc, den, b2, batch_pad, fc1_W, fc1_b, fc2_W, fc2_b)


# speedup vs baseline: 9.3341x; 9.3341x over previous
import functools

import jax
import jax.numpy as jnp
from jax import lax
from jax.experimental import pallas as pl
from jax.experimental.pallas import tpu as pltpu
from jax.experimental.pallas import tpu_sc as plsc

N = 10000
E = 320000
D = 128
H = 128
FC = 256
C = 2
G = 64

NC = 2
NS = 16
NW = NC * NS
L = 16
EPW = E // NW
K = 80
NCH = EPW // K
NP = 10240
RPT = NP // NS

_mesh = plsc.VectorSubcoreMesh(core_axis_name="c", subcore_axis_name="s")


@functools.partial(
    pl.kernel,
    out_type=[
        jax.ShapeDtypeStruct((NC, NP, H), jnp.float32),
        jax.ShapeDtypeStruct((NC, NP), jnp.float32),
    ],
    mesh=_mesh,
    compiler_params=pltpu.CompilerParams(needs_layout_passes=False),
    scratch_types=[
        pltpu.VMEM((K,), jnp.int32),
        pltpu.VMEM((K,), jnp.int32),
        pltpu.VMEM((K, H), jnp.float32),
        pltpu.VMEM((K, H), jnp.float32),
        pltpu.VMEM((K,), jnp.float32),
        pltpu.VMEM((H,), jnp.float32),
        pltpu.VMEM((L, L), jnp.float32),
        pltpu.VMEM((RPT,), jnp.float32),
        pltpu.VMEM_SHARED((NP, H), jnp.float32),
        pltpu.VMEM_SHARED((NP,), jnp.float32),
        pltpu.SemaphoreType.DMA,
        pltpu.SemaphoreType.DMA,
    ],
)
def _edge_pass(xl_hbm, xr_hbm, src_hbm, dst_hbm, att_hbm,
               acc_out, den_out,
               sidx, didx, A, B, W, attv, T, Z, accs, dens, sem1, sem2):
    c = lax.axis_index("c")
    s = lax.axis_index("s")
    wid = s * NC + c
    iota = lax.broadcasted_iota(jnp.int32, (L,), 0)
    zeros16 = jnp.zeros((L,), jnp.float32)

    pltpu.sync_copy(att_hbm, attv)

    def _zero_a(r, _):
        for j in range(H // L):
            A[r, pl.ds(j * L, L)] = zeros16
        return 0
    lax.fori_loop(0, K, _zero_a, 0)

    def _zero_z(i, _):
        Z[pl.ds(i * L, L)] = zeros16
        return 0
    lax.fori_loop(0, RPT // L, _zero_z, 0)

    rbase = s * RPT
    for r8 in range(RPT // K):
        pltpu.sync_copy(A, accs.at[pl.ds(rbase + r8 * K, K)])
    pltpu.sync_copy(Z, dens.at[pl.ds(rbase, RPT)])
    plsc.subcore_barrier()

    def _chunk(g, _):
        ebase = wid * EPW + g * K
        pltpu.sync_copy(src_hbm.at[pl.ds(ebase, K)], sidx)
        pltpu.sync_copy(dst_hbm.at[pl.ds(ebase, K)], didx)
        cp1 = pltpu.async_copy(xl_hbm.at[sidx], A, sem1)
        cp2 = pltpu.async_copy(xr_hbm.at[didx], B, sem2)
        cp1.wait()
        cp2.wait()

        att_chunks = [attv[pl.ds(j * L, L)] for j in range(H // L)]

        def _group(eg, _):
            base = eg * L
            for e in range(L):
                row = base + e
                acc16 = zeros16
                for dd in range(H // L):
                    t = A[row, pl.ds(dd * L, L)] + B[row, pl.ds(dd * L, L)]
                    t = jnp.maximum(t, t * 0.2)
                    acc16 = acc16 + t * att_chunks[dd]
                plsc.store_scatter(
                    T, [iota, jnp.full((L,), e, jnp.int32)], acc16)
            alpha = T[0, :]
            for j in range(1, L):
                alpha = alpha + T[j, :]
            wv = jnp.exp(alpha)
            W[pl.ds(base, L)] = wv
            for e in range(L):
                row = base + e
                we = plsc.load_gather(
                    W, [jnp.full((L,), row, jnp.int32)])
                for dd in range(H // L):
                    A[row, pl.ds(dd * L, L)] = A[row, pl.ds(dd * L, L)] * we
            return 0

        lax.fori_loop(0, K // L, _group, 0)

        pltpu.sync_copy(A, accs.at[didx], add=True)
        pltpu.sync_copy(W, dens.at[didx], add=True)
        return 0

    lax.fori_loop(0, NCH, _chunk, 0)
    plsc.subcore_barrier()

    pltpu.sync_copy(accs.at[pl.ds(rbase, RPT)], acc_out.at[c, pl.ds(rbase, RPT)])
    pltpu.sync_copy(dens.at[pl.ds(rbase, RPT)], den_out.at[c, pl.ds(rbase, RPT)])


def _tc0_body(x_ref, wl_ref, wr_ref, xl_ref, xr_ref):
    xb = x_ref[...]
    xl_ref[...] = jnp.dot(xb, wl_ref[...], preferred_element_type=jnp.float32)
    xr_ref[...] = jnp.dot(xb, wr_ref[...], preferred_element_type=jnp.float32)


def _tc0(x, Wl, Wr):
    return pl.pallas_call(
        _tc0_body,
        out_shape=[jax.ShapeDtypeStruct((N, H), jnp.float32),
                   jax.ShapeDtypeStruct((N, H), jnp.float32)],
    )(x, Wl, Wr)


def _combine(acc_ref, den_ref, b_ref):
    acc = acc_ref[0, :N, :] + acc_ref[1, :N, :]
    den = den_ref[0, :N] + den_ref[1, :N]
    v = acc / (den + 1e-16)[:, None] + b_ref[...][None, :]
    return jnp.where(v > 0.0, v, jnp.exp(v) - 1.0)


def _tcmid_body(acc_ref, den_ref, b_ref, wl_ref, wr_ref, xl_ref, xr_ref):
    h = _combine(acc_ref, den_ref, b_ref)
    xl_ref[...] = jnp.dot(h, wl_ref[...], preferred_element_type=jnp.float32)
    xr_ref[...] = jnp.dot(h, wr_ref[...], preferred_element_type=jnp.float32)


def _tcmid(acc, den, b, Wl, Wr):
    return pl.pallas_call(
        _tcmid_body,
        out_shape=[jax.ShapeDtypeStruct((N, H), jnp.float32),
                   jax.ShapeDtypeStruct((N, H), jnp.float32)],
    )(acc, den, b, Wl, Wr)


def _tcfinal_body(acc_ref, den_ref, b_ref, batch_ref, fc1w_ref, fc1b_ref,
                  fc2w_ref, fc2b_ref, out_ref):
    h = _combine(acc_ref, den_ref, b_ref)
    batch = batch_ref[:N]
    onehot = (batch[:, None] == lax.broadcasted_iota(jnp.int32, (N, G), 1))
    onehot = onehot.astype(jnp.float32)
    sums = jnp.dot(onehot.T, h, preferred_element_type=jnp.float32)
    counts = jnp.sum(onehot, axis=0)
    pooled = sums / jnp.maximum(counts, 1.0)[:, None]
    z = jnp.dot(pooled, fc1w_ref[...], preferred_element_type=jnp.float32)
    z = jnp.maximum(z + fc1b_ref[...][None, :], 0.0)
    z = jnp.dot(z, fc2w_ref[...], preferred_element_type=jnp.float32)
    z = z + fc2b_ref[...][None, :]
    m = jnp.max(z, axis=1, keepdims=True)
    lse = m + jnp.log(jnp.sum(jnp.exp(z - m), axis=1, keepdims=True))
    out_ref[...] = z - lse


def _tcfinal(acc, den, b, batch_pad, fc1_W, fc1_b, fc2_W, fc2_b):
    return pl.pallas_call(
        _tcfinal_body,
        out_shape=jax.ShapeDtypeStruct((G, C), jnp.float32),
    )(acc, den, b, batch_pad, fc1_W, fc1_b, fc2_W, fc2_b)


def kernel(x, edge_index, batch, Wl0, Wr0, att0, b0, Wl1, Wr1, att1, b1,
           Wl2, Wr2, att2, b2, fc1_W, fc1_b, fc2_W, fc2_b):
    src = edge_index[0]
    dst = edge_index[1]
    batch_pad = jnp.pad(batch, (0, NP - N), constant_values=G)

    xl, xr = _tc0(x, Wl0, Wr0)
    acc, den = _edge_pass(xl, xr, src, dst, att0)
    xl, xr = _tcmid(acc, den, b0, Wl1, Wr1)
    acc, den = _edge_pass(xl, xr, src, dst, att1)
    xl, xr = _tcmid(acc, den, b1, Wl2, Wr2)
    acc, den = _edge_pass(xl, xr, src, dst, att2)
    return _tcfinal(acc, den, b2, batch_pad, fc1_W, fc1_b, fc2_W, fc2_b)


# speedup vs baseline: 11.6481x; 1.2479x over previous
import functools

import jax
import jax.numpy as jnp
from jax import lax
from jax.experimental import pallas as pl
from jax.experimental.pallas import tpu as pltpu
from jax.experimental.pallas import tpu_sc as plsc

N = 10000
E = 320000
D = 128
H = 128
FC = 256
C = 2
G = 64

NC = 2
NS = 16
NW = NC * NS
L = 16
EPW = E // NW
K = 80
NCH = EPW // K
NB = 2
NP = 10240
RPT = NP // NS

_mesh = plsc.VectorSubcoreMesh(core_axis_name="c", subcore_axis_name="s")

_scratch = []
for _ in range(NB):
    _scratch += [
        pltpu.VMEM((K,), jnp.int32),
        pltpu.VMEM((K,), jnp.int32),
        pltpu.VMEM((K, H), jnp.float32),
        pltpu.VMEM((K, H), jnp.float32),
        pltpu.VMEM((K,), jnp.float32),
        pltpu.SemaphoreType.DMA,
    ]
_scratch += [
    pltpu.VMEM((H,), jnp.float32),
    pltpu.VMEM((L, L), jnp.float32),
    pltpu.VMEM((RPT,), jnp.float32),
    pltpu.VMEM_SHARED((NP, H), jnp.float32),
    pltpu.VMEM_SHARED((NP,), jnp.float32),
]


@functools.partial(
    pl.kernel,
    out_type=[
        jax.ShapeDtypeStruct((NC, NP, H), jnp.float32),
        jax.ShapeDtypeStruct((NC, NP), jnp.float32),
    ],
    mesh=_mesh,
    compiler_params=pltpu.CompilerParams(needs_layout_passes=False),
    scratch_types=_scratch,
)
def _edge_pass(xl_hbm, xr_hbm, src_hbm, dst_hbm, att_hbm,
               acc_out, den_out, *scr):
    bufs = [scr[6 * b:6 * b + 6] for b in range(NB)]
    attv, T, Z, accs, dens = scr[6 * NB:]
    c = lax.axis_index("c")
    s = lax.axis_index("s")
    wid = s * NC + c
    iota = lax.broadcasted_iota(jnp.int32, (L,), 0)
    zeros16 = jnp.zeros((L,), jnp.float32)
    A0 = bufs[0][2]

    pltpu.sync_copy(att_hbm, attv)

    def _zero_a(r, _):
        for j in range(H // L):
            A0[r, pl.ds(j * L, L)] = zeros16
        return 0
    lax.fori_loop(0, K, _zero_a, 0)

    def _zero_z(i, _):
        Z[pl.ds(i * L, L)] = zeros16
        return 0
    lax.fori_loop(0, RPT // L, _zero_z, 0)

    rbase = s * RPT
    for r8 in range(RPT // K):
        pltpu.sync_copy(A0, accs.at[pl.ds(rbase + r8 * K, K)])
    pltpu.sync_copy(Z, dens.at[pl.ds(rbase, RPT)])
    plsc.subcore_barrier()

    def _issue(g, b):
        sidx, didx, A, B, _, sem = bufs[b]
        ebase = wid * EPW + g * K
        pltpu.sync_copy(src_hbm.at[pl.ds(ebase, K)], sidx)
        pltpu.sync_copy(dst_hbm.at[pl.ds(ebase, K)], didx)
        pltpu.async_copy(xl_hbm.at[sidx], A, sem)
        pltpu.async_copy(xr_hbm.at[didx], B, sem)

    def _process(b):
        sidx, didx, A, B, W, sem = bufs[b]
        pltpu.make_async_copy(xl_hbm.at[sidx], A, sem).wait()
        pltpu.make_async_copy(xr_hbm.at[didx], B, sem).wait()

        att_chunks = [attv[pl.ds(j * L, L)] for j in range(H // L)]

        def _group(eg, _):
            base = eg * L
            for e in range(L):
                row = base + e
                acc16 = zeros16
                for dd in range(H // L):
                    t = A[row, pl.ds(dd * L, L)] + B[row, pl.ds(dd * L, L)]
                    t = jnp.maximum(t, t * 0.2)
                    acc16 = acc16 + t * att_chunks[dd]
                plsc.store_scatter(
                    T, [iota, jnp.full((L,), e, jnp.int32)], acc16)
            alpha = T[0, :]
            for j in range(1, L):
                alpha = alpha + T[j, :]
            wv = jnp.exp(alpha)
            W[pl.ds(base, L)] = wv
            for e in range(L):
                row = base + e
                we = plsc.load_gather(
                    W, [jnp.full((L,), row, jnp.int32)])
                for dd in range(H // L):
                    A[row, pl.ds(dd * L, L)] = A[row, pl.ds(dd * L, L)] * we
            return 0

        lax.fori_loop(0, K // L, _group, 0)

        pltpu.sync_copy(A, accs.at[didx], add=True)
        pltpu.sync_copy(W, dens.at[didx], add=True)

    for b in range(NB):
        _issue(b, b)

    def _iter(i, _):
        for b in range(NB):
            g = i * NB + b
            _process(b)

            @pl.when(g + NB < NCH)
            def _():
                _issue(g + NB, b)
        return 0

    lax.fori_loop(0, NCH // NB, _iter, 0)
    for b in range(NCH % NB):
        _process(b)
    plsc.subcore_barrier()

    pltpu.sync_copy(accs.at[pl.ds(rbase, RPT)], acc_out.at[c, pl.ds(rbase, RPT)])
    pltpu.sync_copy(dens.at[pl.ds(rbase, RPT)], den_out.at[c, pl.ds(rbase, RPT)])


def _tc0_body(x_ref, wl_ref, wr_ref, xl_ref, xr_ref):
    xb = x_ref[...]
    xl_ref[...] = jnp.dot(xb, wl_ref[...], preferred_element_type=jnp.float32)
    xr_ref[...] = jnp.dot(xb, wr_ref[...], preferred_element_type=jnp.float32)


def _tc0(x, Wl, Wr):
    return pl.pallas_call(
        _tc0_body,
        out_shape=[jax.ShapeDtypeStruct((N, H), jnp.float32),
                   jax.ShapeDtypeStruct((N, H), jnp.float32)],
    )(x, Wl, Wr)


def _combine(acc_ref, den_ref, b_ref):
    acc = acc_ref[0, :N, :] + acc_ref[1, :N, :]
    den = den_ref[0, :N] + den_ref[1, :N]
    v = acc / (den + 1e-16)[:, None] + b_ref[...][None, :]
    return jnp.where(v > 0.0, v, jnp.exp(v) - 1.0)


def _tcmid_body(acc_ref, den_ref, b_ref, wl_ref, wr_ref, xl_ref, xr_ref):
    h = _combine(acc_ref, den_ref, b_ref)
    xl_ref[...] = jnp.dot(h, wl_ref[...], preferred_element_type=jnp.float32)
    xr_ref[...] = jnp.dot(h, wr_ref[...], preferred_element_type=jnp.float32)


def _tcmid(acc, den, b, Wl, Wr):
    return pl.pallas_call(
        _tcmid_body,
        out_shape=[jax.ShapeDtypeStruct((N, H), jnp.float32),
                   jax.ShapeDtypeStruct((N, H), jnp.float32)],
    )(acc, den, b, Wl, Wr)


def _tcfinal_body(acc_ref, den_ref, b_ref, batch_ref, fc1w_ref, fc1b_ref,
                  fc2w_ref, fc2b_ref, out_ref):
    h = _combine(acc_ref, den_ref, b_ref)
    batch = batch_ref[:N]
    onehot = (batch[:, None] == lax.broadcasted_iota(jnp.int32, (N, G), 1))
    onehot = onehot.astype(jnp.float32)
    sums = jnp.dot(onehot.T, h, preferred_element_type=jnp.float32)
    counts = jnp.sum(onehot, axis=0)
    pooled = sums / jnp.maximum(counts, 1.0)[:, None]
    z = jnp.dot(pooled, fc1w_ref[...], preferred_element_type=jnp.float32)
    z = jnp.maximum(z + fc1b_ref[...][None, :], 0.0)
    z = jnp.dot(z, fc2w_ref[...], preferred_element_type=jnp.float32)
    z = z + fc2b_ref[...][None, :]
    m = jnp.max(z, axis=1, keepdims=True)
    lse = m + jnp.log(jnp.sum(jnp.exp(z - m), axis=1, keepdims=True))
    out_ref[...] = z - lse


def _tcfinal(acc, den, b, batch_pad, fc1_W, fc1_b, fc2_W, fc2_b):
    return pl.pallas_call(
        _tcfinal_body,
        out_shape=jax.ShapeDtypeStruct((G, C), jnp.float32),
    )(acc, den, b, batch_pad, fc1_W, fc1_b, fc2_W, fc2_b)


def kernel(x, edge_index, batch, Wl0, Wr0, att0, b0, Wl1, Wr1, att1, b1,
           Wl2, Wr2, att2, b2, fc1_W, fc1_b, fc2_W, fc2_b):
    src = edge_index[0]
    dst = edge_index[1]
    batch_pad = jnp.pad(batch, (0, NP - N), constant_values=G)

    xl, xr = _tc0(x, Wl0, Wr0)
    acc, den = _edge_pass(xl, xr, src, dst, att0)
    xl, xr = _tcmid(acc, den, b0, Wl1, Wr1)
    acc, den = _edge_pass(xl, xr, src, dst, att1)
    xl, xr = _tcmid(acc, den, b1, Wl2, Wr2)
    acc, den = _edge_pass(xl, xr, src, dst, att2)
    return _tcfinal(acc, den, b2, batch_pad, fc1_W, fc1_b, fc2_W, fc2_b)


# speedup vs baseline: 11.7242x; 1.0065x over previous
import functools

import jax
import jax.numpy as jnp
from jax import lax
from jax.experimental import pallas as pl
from jax.experimental.pallas import tpu as pltpu
from jax.experimental.pallas import tpu_sc as plsc

N = 10000
E = 320000
D = 128
H = 128
FC = 256
C = 2
G = 64

NC = 2
NS = 16
NW = NC * NS
L = 16
EPW = E // NW
K = 80
NCH = EPW // K
NB = 2
NP = 10240
RPT = NP // NS

_mesh = plsc.VectorSubcoreMesh(core_axis_name="c", subcore_axis_name="s")

_NSL = 8
_scratch = []
for _ in range(NB):
    _scratch += [
        pltpu.VMEM((K,), jnp.int32),
        pltpu.VMEM((K,), jnp.int32),
        pltpu.VMEM((K, H), jnp.float32),
        pltpu.VMEM((K, H), jnp.float32),
        pltpu.VMEM((K,), jnp.float32),
        pltpu.VMEM((K,), jnp.float32),
        pltpu.VMEM((K,), jnp.float32),
        pltpu.SemaphoreType.DMA,
    ]
_scratch += [
    pltpu.VMEM((H,), jnp.float32),
    pltpu.VMEM((L, L), jnp.float32),
    pltpu.VMEM((RPT,), jnp.float32),
    pltpu.VMEM_SHARED((NP, H), jnp.float32),
    pltpu.VMEM_SHARED((NP,), jnp.float32),
]


@functools.partial(
    pl.kernel,
    out_type=[
        jax.ShapeDtypeStruct((NC, NP, H), jnp.float32),
        jax.ShapeDtypeStruct((NC, NP), jnp.float32),
    ],
    mesh=_mesh,
    compiler_params=pltpu.CompilerParams(needs_layout_passes=False),
    scratch_types=_scratch,
)
def _edge_pass(xl_hbm, xr_hbm, src_hbm, dst_hbm, att_hbm, u_hbm, v_hbm,
               acc_out, den_out, *scr):
    bufs = [scr[_NSL * b:_NSL * b + _NSL] for b in range(NB)]
    attv, T, Z, accs, dens = scr[_NSL * NB:]
    c = lax.axis_index("c")
    s = lax.axis_index("s")
    wid = s * NC + c
    iota = lax.broadcasted_iota(jnp.int32, (L,), 0)
    zeros16 = jnp.zeros((L,), jnp.float32)
    A0 = bufs[0][2]

    pltpu.sync_copy(att_hbm, attv)

    def _zero_a(r, _):
        for j in range(H // L):
            A0[r, pl.ds(j * L, L)] = zeros16
        return 0
    lax.fori_loop(0, K, _zero_a, 0)

    def _zero_z(i, _):
        Z[pl.ds(i * L, L)] = zeros16
        return 0
    lax.fori_loop(0, RPT // L, _zero_z, 0)

    rbase = s * RPT
    for r8 in range(RPT // K):
        pltpu.sync_copy(A0, accs.at[pl.ds(rbase + r8 * K, K)])
    pltpu.sync_copy(Z, dens.at[pl.ds(rbase, RPT)])
    plsc.subcore_barrier()

    def _issue(g, b):
        sidx, didx, A, B, _, Ub, Vb, sem = bufs[b]
        ebase = wid * EPW + g * K
        pltpu.sync_copy(src_hbm.at[pl.ds(ebase, K)], sidx)
        pltpu.sync_copy(dst_hbm.at[pl.ds(ebase, K)], didx)
        pltpu.async_copy(xl_hbm.at[sidx], A, sem)
        pltpu.async_copy(xr_hbm.at[didx], B, sem)
        pltpu.async_copy(u_hbm.at[sidx], Ub, sem)
        pltpu.async_copy(v_hbm.at[didx], Vb, sem)

    def _process(b):
        sidx, didx, A, B, W, Ub, Vb, sem = bufs[b]
        pltpu.make_async_copy(xl_hbm.at[sidx], A, sem).wait()
        pltpu.make_async_copy(xr_hbm.at[didx], B, sem).wait()
        pltpu.make_async_copy(u_hbm.at[sidx], Ub, sem).wait()
        pltpu.make_async_copy(v_hbm.at[didx], Vb, sem).wait()

        att_chunks = [attv[pl.ds(j * L, L)] for j in range(H // L)]

        def _group(eg, _):
            base = eg * L
            for e in range(L):
                row = base + e
                t = A[row, pl.ds(0, L)] + B[row, pl.ds(0, L)]
                acc16 = jnp.maximum(t, zeros16) * att_chunks[0]
                for dd in range(1, H // L):
                    t = A[row, pl.ds(dd * L, L)] + B[row, pl.ds(dd * L, L)]
                    acc16 = acc16 + jnp.maximum(t, zeros16) * att_chunks[dd]
                plsc.store_scatter(
                    T, [iota, jnp.full((L,), e, jnp.int32)], acc16)
            alpha = Ub[pl.ds(base, L)] + Vb[pl.ds(base, L)]
            for j in range(L):
                alpha = alpha + T[j, :]
            wv = jnp.exp(alpha)
            W[pl.ds(base, L)] = wv
            for e in range(L):
                row = base + e
                we = plsc.load_gather(
                    W, [jnp.full((L,), row, jnp.int32)])
                for dd in range(H // L):
                    A[row, pl.ds(dd * L, L)] = A[row, pl.ds(dd * L, L)] * we
            return 0

        lax.fori_loop(0, K // L, _group, 0)

        pltpu.sync_copy(A, accs.at[didx], add=True)
        pltpu.sync_copy(W, dens.at[didx], add=True)

    for b in range(NB):
        _issue(b, b)

    def _iter(i, _):
        for b in range(NB):
            g = i * NB + b
            _process(b)

            @pl.when(g + NB < NCH)
            def _():
                _issue(g + NB, b)
        return 0

    lax.fori_loop(0, NCH // NB, _iter, 0)
    for b in range(NCH % NB):
        _process(b)
    plsc.subcore_barrier()

    pltpu.sync_copy(accs.at[pl.ds(rbase, RPT)], acc_out.at[c, pl.ds(rbase, RPT)])
    pltpu.sync_copy(dens.at[pl.ds(rbase, RPT)], den_out.at[c, pl.ds(rbase, RPT)])


def _tc0_body(x_ref, wl_ref, wr_ref, att_ref, xl_ref, xr_ref, u_ref, v_ref):
    xb = x_ref[...]
    xl = jnp.dot(xb, wl_ref[...], preferred_element_type=jnp.float32)
    xr = jnp.dot(xb, wr_ref[...], preferred_element_type=jnp.float32)
    xl_ref[...] = xl
    xr_ref[...] = xr
    a02 = att_ref[...] * 0.2
    u_ref[...] = jnp.dot(xl, a02, preferred_element_type=jnp.float32)
    v_ref[...] = jnp.dot(xr, a02, preferred_element_type=jnp.float32)


def _tc0(x, Wl, Wr, att):
    return pl.pallas_call(
        _tc0_body,
        out_shape=[jax.ShapeDtypeStruct((N, H), jnp.float32),
                   jax.ShapeDtypeStruct((N, H), jnp.float32),
                   jax.ShapeDtypeStruct((N,), jnp.float32),
                   jax.ShapeDtypeStruct((N,), jnp.float32)],
    )(x, Wl, Wr, att)


def _combine(acc_ref, den_ref, b_ref):
    acc = acc_ref[0, :N, :] + acc_ref[1, :N, :]
    den = den_ref[0, :N] + den_ref[1, :N]
    v = acc / (den + 1e-16)[:, None] + b_ref[...][None, :]
    return jnp.where(v > 0.0, v, jnp.exp(v) - 1.0)


def _tcmid_body(acc_ref, den_ref, b_ref, wl_ref, wr_ref, att_ref,
                xl_ref, xr_ref, u_ref, v_ref):
    h = _combine(acc_ref, den_ref, b_ref)
    xl = jnp.dot(h, wl_ref[...], preferred_element_type=jnp.float32)
    xr = jnp.dot(h, wr_ref[...], preferred_element_type=jnp.float32)
    xl_ref[...] = xl
    xr_ref[...] = xr
    a02 = att_ref[...] * 0.2
    u_ref[...] = jnp.dot(xl, a02, preferred_element_type=jnp.float32)
    v_ref[...] = jnp.dot(xr, a02, preferred_element_type=jnp.float32)


def _tcmid(acc, den, b, Wl, Wr, att):
    return pl.pallas_call(
        _tcmid_body,
        out_shape=[jax.ShapeDtypeStruct((N, H), jnp.float32),
                   jax.ShapeDtypeStruct((N, H), jnp.float32),
                   jax.ShapeDtypeStruct((N,), jnp.float32),
                   jax.ShapeDtypeStruct((N,), jnp.float32)],
    )(acc, den, b, Wl, Wr, att)


def _tcfinal_body(acc_ref, den_ref, b_ref, batch_ref, fc1w_ref, fc1b_ref,
                  fc2w_ref, fc2b_ref, out_ref):
    h = _combine(acc_ref, den_ref, b_ref)
    batch = batch_ref[:N]
    onehot = (batch[:, None] == lax.broadcasted_iota(jnp.int32, (N, G), 1))
    onehot = onehot.astype(jnp.float32)
    sums = jnp.dot(onehot.T, h, preferred_element_type=jnp.float32)
    counts = jnp.sum(onehot, axis=0)
    pooled = sums / jnp.maximum(counts, 1.0)[:, None]
    z = jnp.dot(pooled, fc1w_ref[...], preferred_element_type=jnp.float32)
    z = jnp.maximum(z + fc1b_ref[...][None, :], 0.0)
    z = jnp.dot(z, fc2w_ref[...], preferred_element_type=jnp.float32)
    z = z + fc2b_ref[...][None, :]
    m = jnp.max(z, axis=1, keepdims=True)
    lse = m + jnp.log(jnp.sum(jnp.exp(z - m), axis=1, keepdims=True))
    out_ref[...] = z - lse


def _tcfinal(acc, den, b, batch_pad, fc1_W, fc1_b, fc2_W, fc2_b):
    return pl.pallas_call(
        _tcfinal_body,
        out_shape=jax.ShapeDtypeStruct((G, C), jnp.float32),
    )(acc, den, b, batch_pad, fc1_W, fc1_b, fc2_W, fc2_b)


def kernel(x, edge_index, batch, Wl0, Wr0, att0, b0, Wl1, Wr1, att1, b1,
           Wl2, Wr2, att2, b2, fc1_W, fc1_b, fc2_W, fc2_b):
    src = edge_index[0]
    dst = edge_index[1]
    batch_pad = jnp.pad(batch, (0, NP - N), constant_values=G)

    a0s = att0 * 0.8
    a1s = att1 * 0.8
    a2s = att2 * 0.8

    xl, xr, u, v = _tc0(x, Wl0, Wr0, att0)
    acc, den = _edge_pass(xl, xr, src, dst, a0s, u, v)
    xl, xr, u, v = _tcmid(acc, den, b0, Wl1, Wr1, att1)
    acc, den = _edge_pass(xl, xr, src, dst, a1s, u, v)
    xl, xr, u, v = _tcmid(acc, den, b1, Wl2, Wr2, att2)
    acc, den = _edge_pass(xl, xr, src, dst, a2s, u, v)
    return _tcfinal(acc, den, b2, batch_pad, fc1_W, fc1_b, fc2_W, fc2_b)


# speedup vs baseline: 14.4063x; 1.2288x over previous
import functools

import jax
import jax.numpy as jnp
from jax import lax
from jax.experimental import pallas as pl
from jax.experimental.pallas import tpu as pltpu
from jax.experimental.pallas import tpu_sc as plsc

N = 10000
E = 320000
D = 128
H = 128
FC = 256
C = 2
G = 64

NC = 2
NS = 16
NW = NC * NS
L = 16
EPW = E // NW
K = 80
NCH = EPW // K
NB = 2
NP = 10240
RPT = NP // NS

_mesh = plsc.VectorSubcoreMesh(core_axis_name="c", subcore_axis_name="s")

_NSL = 9
_scratch = []
for _ in range(NB):
    _scratch += [
        pltpu.VMEM((K, H), jnp.float32),
        pltpu.VMEM((K, H), jnp.float32),
        pltpu.VMEM((K,), jnp.float32),
        pltpu.VMEM((K,), jnp.float32),
        pltpu.VMEM((K,), jnp.float32),
        pltpu.VMEM((K,), jnp.int32),
        pltpu.VMEM((K,), jnp.int32),
        pltpu.SemaphoreType.DMA,
        pltpu.SemaphoreType.DMA,
    ]
_scratch += [
    pltpu.VMEM((H,), jnp.float32),
    pltpu.VMEM((L, L), jnp.float32),
    pltpu.VMEM((RPT,), jnp.float32),
    pltpu.VMEM_SHARED((NP, H), jnp.float32),
    pltpu.VMEM_SHARED((NP,), jnp.float32),
]


@functools.partial(
    pl.kernel,
    out_type=[
        jax.ShapeDtypeStruct((NC, NP, H), jnp.float32),
        jax.ShapeDtypeStruct((NC, NP), jnp.float32),
    ],
    mesh=_mesh,
    compiler_params=pltpu.CompilerParams(needs_layout_passes=False),
    scratch_types=_scratch,
)
def _edge_pass(xl_hbm, xr_hbm, src_hbm, dst_hbm, att_hbm, u_hbm, v_hbm,
               acc_out, den_out, *scr):
    bufs = [scr[_NSL * b:_NSL * b + _NSL] for b in range(NB)]
    attv, T, Z, accs, dens = scr[_NSL * NB:]
    c = lax.axis_index("c")
    s = lax.axis_index("s")
    wid = s * NC + c
    iota = lax.broadcasted_iota(jnp.int32, (L,), 0)
    zeros16 = jnp.zeros((L,), jnp.float32)
    A0 = bufs[0][0]

    pltpu.sync_copy(att_hbm, attv)

    def _zero_a(r, _):
        for j in range(H // L):
            A0[r, pl.ds(j * L, L)] = zeros16
        return 0
    lax.fori_loop(0, K, _zero_a, 0)

    def _zero_z(i, _):
        Z[pl.ds(i * L, L)] = zeros16
        return 0
    lax.fori_loop(0, RPT // L, _zero_z, 0)

    rbase = s * RPT
    for r8 in range(RPT // K):
        pltpu.sync_copy(A0, accs.at[pl.ds(rbase + r8 * K, K)])
    pltpu.sync_copy(Z, dens.at[pl.ds(rbase, RPT)])
    plsc.subcore_barrier()

    def _issue_idx(g, b):
        sidx, didx, isem = bufs[b][5], bufs[b][6], bufs[b][8]
        ebase = wid * EPW + g * K
        pltpu.async_copy(src_hbm.at[pl.ds(ebase, K)], sidx, isem)
        pltpu.async_copy(dst_hbm.at[pl.ds(ebase, K)], didx, isem)

    def _issue(g, b):
        A, B, _, Ub, Vb, sidx, didx, sem, isem = bufs[b]
        ebase = wid * EPW + g * K
        pltpu.make_async_copy(src_hbm.at[pl.ds(ebase, K)], sidx, isem).wait()
        pltpu.make_async_copy(dst_hbm.at[pl.ds(ebase, K)], didx, isem).wait()
        pltpu.async_copy(xl_hbm.at[sidx], A, sem)
        pltpu.async_copy(xr_hbm.at[didx], B, sem)
        pltpu.async_copy(u_hbm.at[sidx], Ub, sem)
        pltpu.async_copy(v_hbm.at[didx], Vb, sem)

    def _process(g, b):
        A, B, W, Ub, Vb, sidx, didx, sem, isem = bufs[b]
        pltpu.make_async_copy(xl_hbm.at[sidx], A, sem).wait()
        pltpu.make_async_copy(xr_hbm.at[didx], B, sem).wait()
        pltpu.make_async_copy(u_hbm.at[sidx], Ub, sem).wait()
        pltpu.make_async_copy(v_hbm.at[didx], Vb, sem).wait()
        @pl.when(g + NB < NCH)
        def _():
            _issue_idx(g + NB, b)

        att_chunks = [attv[pl.ds(j * L, L)] for j in range(H // L)]

        def _group(eg, _):
            base = eg * L
            for e in range(L):
                row = base + e
                t = A[row, pl.ds(0, L)] + B[row, pl.ds(0, L)]
                acc16 = jnp.maximum(t, zeros16) * att_chunks[0]
                for dd in range(1, H // L):
                    t = A[row, pl.ds(dd * L, L)] + B[row, pl.ds(dd * L, L)]
                    acc16 = acc16 + jnp.maximum(t, zeros16) * att_chunks[dd]
                plsc.store_scatter(
                    T, [iota, jnp.full((L,), e, jnp.int32)], acc16)
            alpha = Ub[pl.ds(base, L)] + Vb[pl.ds(base, L)]
            for j in range(L):
                alpha = alpha + T[j, :]
            wv = jnp.exp(alpha)
            W[pl.ds(base, L)] = wv
            for e in range(L):
                row = base + e
                we = plsc.load_gather(
                    W, [jnp.full((L,), row, jnp.int32)])
                for dd in range(H // L):
                    A[row, pl.ds(dd * L, L)] = A[row, pl.ds(dd * L, L)] * we
            return 0

        lax.fori_loop(0, K // L, _group, 0)

        pltpu.sync_copy(A, accs.at[didx], add=True)
        pltpu.sync_copy(W, dens.at[didx], add=True)

    for b in range(NB):
        _issue_idx(b, b)
    for b in range(NB):
        _issue(b, b)

    def _iter(i, _):
        for b in range(NB):
            g = i * NB + b
            _process(g, b)

            @pl.when(g + NB < NCH)
            def _():
                _issue(g + NB, b)
        return 0

    lax.fori_loop(0, NCH // NB, _iter, 0)
    for b in range(NCH % NB):
        _process(NCH - (NCH % NB) + b, b)
    plsc.subcore_barrier()

    pltpu.sync_copy(accs.at[pl.ds(rbase, RPT)], acc_out.at[c, pl.ds(rbase, RPT)])
    pltpu.sync_copy(dens.at[pl.ds(rbase, RPT)], den_out.at[c, pl.ds(rbase, RPT)])


def _tc0_body(x_ref, wl_ref, wr_ref, att_ref, xl_ref, xr_ref, u_ref, v_ref):
    xb = x_ref[...]
    xl = jnp.dot(xb, wl_ref[...], preferred_element_type=jnp.float32)
    xr = jnp.dot(xb, wr_ref[...], preferred_element_type=jnp.float32)
    xl_ref[...] = xl
    xr_ref[...] = xr
    a02 = att_ref[...] * 0.2
    u_ref[...] = jnp.dot(xl, a02, preferred_element_type=jnp.float32)
    v_ref[...] = jnp.dot(xr, a02, preferred_element_type=jnp.float32)


def _tc0(x, Wl, Wr, att):
    return pl.pallas_call(
        _tc0_body,
        out_shape=[jax.ShapeDtypeStruct((N, H), jnp.float32),
                   jax.ShapeDtypeStruct((N, H), jnp.float32),
                   jax.ShapeDtypeStruct((N,), jnp.float32),
                   jax.ShapeDtypeStruct((N,), jnp.float32)],
    )(x, Wl, Wr, att)


def _combine(acc_ref, den_ref, b_ref):
    acc = acc_ref[0, :N, :] + acc_ref[1, :N, :]
    den = den_ref[0, :N] + den_ref[1, :N]
    v = acc / (den + 1e-16)[:, None] + b_ref[...][None, :]
    return jnp.where(v > 0.0, v, jnp.exp(v) - 1.0)


def _tcmid_body(acc_ref, den_ref, b_ref, wl_ref, wr_ref, att_ref,
                xl_ref, xr_ref, u_ref, v_ref):
    h = _combine(acc_ref, den_ref, b_ref)
    xl = jnp.dot(h, wl_ref[...], preferred_element_type=jnp.float32)
    xr = jnp.dot(h, wr_ref[...], preferred_element_type=jnp.float32)
    xl_ref[...] = xl
    xr_ref[...] = xr
    a02 = att_ref[...] * 0.2
    u_ref[...] = jnp.dot(xl, a02, preferred_element_type=jnp.float32)
    v_ref[...] = jnp.dot(xr, a02, preferred_element_type=jnp.float32)


def _tcmid(acc, den, b, Wl, Wr, att):
    return pl.pallas_call(
        _tcmid_body,
        out_shape=[jax.ShapeDtypeStruct((N, H), jnp.float32),
                   jax.ShapeDtypeStruct((N, H), jnp.float32),
                   jax.ShapeDtypeStruct((N,), jnp.float32),
                   jax.ShapeDtypeStruct((N,), jnp.float32)],
    )(acc, den, b, Wl, Wr, att)


def _tcfinal_body(acc_ref, den_ref, b_ref, batch_ref, fc1w_ref, fc1b_ref,
                  fc2w_ref, fc2b_ref, out_ref):
    h = _combine(acc_ref, den_ref, b_ref)
    batch = batch_ref[:N]
    onehot = (batch[:, None] == lax.broadcasted_iota(jnp.int32, (N, G), 1))
    onehot = onehot.astype(jnp.float32)
    sums = jnp.dot(onehot.T, h, preferred_element_type=jnp.float32)
    counts = jnp.sum(onehot, axis=0)
    pooled = sums / jnp.maximum(counts, 1.0)[:, None]
    z = jnp.dot(pooled, fc1w_ref[...], preferred_element_type=jnp.float32)
    z = jnp.maximum(z + fc1b_ref[...][None, :], 0.0)
    z = jnp.dot(z, fc2w_ref[...], preferred_element_type=jnp.float32)
    z = z + fc2b_ref[...][None, :]
    m = jnp.max(z, axis=1, keepdims=True)
    lse = m + jnp.log(jnp.sum(jnp.exp(z - m), axis=1, keepdims=True))
    out_ref[...] = z - lse


def _tcfinal(acc, den, b, batch_pad, fc1_W, fc1_b, fc2_W, fc2_b):
    return pl.pallas_call(
        _tcfinal_body,
        out_shape=jax.ShapeDtypeStruct((G, C), jnp.float32),
    )(acc, den, b, batch_pad, fc1_W, fc1_b, fc2_W, fc2_b)


def kernel(x, edge_index, batch, Wl0, Wr0, att0, b0, Wl1, Wr1, att1, b1,
           Wl2, Wr2, att2, b2, fc1_W, fc1_b, fc2_W, fc2_b):
    src = edge_index[0]
    dst = edge_index[1]
    batch_pad = jnp.pad(batch, (0, NP - N), constant_values=G)

    a0s = att0 * 0.8
    a1s = att1 * 0.8
    a2s = att2 * 0.8

    xl, xr, u, v = _tc0(x, Wl0, Wr0, att0)
    acc, den = _edge_pass(xl, xr, src, dst, a0s, u, v)
    xl, xr, u, v = _tcmid(acc, den, b0, Wl1, Wr1, att1)
    acc, den = _edge_pass(xl, xr, src, dst, a1s, u, v)
    xl, xr, u, v = _tcmid(acc, den, b1, Wl2, Wr2, att2)
    acc, den = _edge_pass(xl, xr, src, dst, a2s, u, v)
    return _tcfinal(acc, den, b2, batch_pad, fc1_W, fc1_b, fc2_W, fc2_b)
